# Initial kernel scaffold; baseline (speedup 1.0000x reference)
import jax
import jax.numpy as jnp
from jax.experimental import pallas as pl


def kernel(x, keys_weight, values_weight):
    raise NotImplementedError("write your pallas kernel here")


# speedup vs baseline: 14.4120x; 14.4120x over previous
import functools

import jax
import jax.numpy as jnp
from jax import lax
from jax.experimental import pallas as pl
from jax.experimental.pallas import tpu as pltpu
from jax.experimental.pallas import tpu_sc as plsc

_DIMS = 1000
_DPAD = 1024
_NWORDS = _DPAD // 32
_LEVELS = 100
_SIZE = 128
_BATCH = 1024
_NTILES = 32
_BPT = _BATCH // _NTILES
_LANES = 16


def _csa(a, b, c):
    t = a ^ b
    return t ^ c, (a & b) | (t & c)


def _accumulate_group(carry, planes):
    ones, twos, fours, eights, s16, s32, s64, s128 = carry
    v0, v1, v2, v3, v4, v5, v6, v7 = planes
    s1, c1 = _csa(v0, v1, v2)
    s2, c2 = _csa(v3, v4, v5)
    s3, c3 = _csa(v6, v7, s1)
    ones, c4 = _csa(s2, s3, ones)
    s5, c5 = _csa(c1, c2, c3)
    twos, c6 = _csa(s5, c4, twos)
    fours, c7 = _csa(c5, c6, fours)
    eights, c8 = eights ^ c7, eights & c7
    s16, c9 = s16 ^ c8, s16 & c8
    s32, c10 = s32 ^ c9, s32 & c9
    s64, c11 = s64 ^ c10, s64 & c10
    s128 = s128 ^ c11
    return ones, twos, fours, eights, s16, s32, s64, s128


def _sc_encode_body(x_hbm, kinv_hbm, vbit_hbm, out_hbm,
                    x_v, idx_v, kinv_v, vbit_v, outstage_v):
    wid = lax.axis_index("s") * 2 + lax.axis_index("c")
    _tile_work(wid, x_hbm, kinv_hbm, vbit_hbm, out_hbm,
               x_v, idx_v, kinv_v, vbit_v, outstage_v)


def _tile_work(wid, x_hbm, kinv_hbm, vbit_hbm, out_hbm,
               x_v, idx_v, kinv_v, vbit_v, outstage_v):
    base = wid * _BPT
    lanes = lax.iota(jnp.int32, _LANES)

    pltpu.sync_copy(x_hbm.at[pl.ds(base, _BPT)], x_v)
    pltpu.sync_copy(kinv_hbm, kinv_v)
    pltpu.sync_copy(vbit_hbm, vbit_v)

    @pl.loop(0, _BPT)
    def _quant(b):
        for c in range(_SIZE // _LANES):
            xv = x_v[b, pl.ds(c * _LANES, _LANES)]
            y = jnp.clip(xv, 0.0, 1.0) * jnp.float32(_LEVELS - 1)
            t0 = y.astype(jnp.int32)
            r = y - t0.astype(jnp.float32)
            up = (r > 0.5) | ((r == 0.5) & ((t0 & 1) == 1))
            idx_v[b, pl.ds(c * _LANES, _LANES)] = t0 + jnp.where(
                up, jnp.int32(1), jnp.int32(0))

    zero = jnp.zeros((_LANES,), jnp.int32)

    @pl.loop(0, _BPT)
    def _row(b):
        init = (zero,) * 16

        @pl.loop(0, _SIZE // _LANES, init_carry=init)
        def _groups(gg, carry):
            iv = idx_v[b, pl.ds(gg * _LANES, _LANES)]
            cols = (carry[:8], carry[8:])
            new_cols = []
            for col in range(2):
                planes = []
                for u in range(_LANES):
                    l = iv[u]
                    vrow = vbit_v[l, pl.ds(col * _LANES, _LANES)]
                    krow = kinv_v[gg * _LANES + u, pl.ds(col * _LANES, _LANES)]
                    planes.append(vrow ^ krow)
                acc = _accumulate_group(cols[col], tuple(planes[:8]))
                acc = _accumulate_group(acc, tuple(planes[8:]))
                new_cols.append(acc)
            return tuple(new_cols[0]) + tuple(new_cols[1])

        acc = _groups
        for col in range(2):
            ones, twos, fours, eights, s16, s32, s64, s128 = acc[col * 8:col * 8 + 8]
            low_any = ones | twos | fours | eights | s16 | s32
            ge = s128 | (s64 & low_any)
            for k in range(_LANES):
                wv = jnp.full((_LANES,), ge[k], jnp.int32)
                lo = (wv >> lanes) & 1
                hi = (wv >> (lanes + 16)) & 1
                ww = col * _LANES + k
                outstage_v[b, pl.ds(ww * 32, _LANES)] = jnp.where(
                    lo == 1, jnp.float32(1.0), jnp.float32(-1.0))
                outstage_v[b, pl.ds(ww * 32 + 16, _LANES)] = jnp.where(
                    hi == 1, jnp.float32(1.0), jnp.float32(-1.0))

    pltpu.sync_copy(outstage_v, out_hbm.at[pl.ds(base, _BPT)])


@functools.lru_cache(maxsize=None)
def _sc_encode():
    return functools.partial(
        pl.kernel,
        out_type=jax.ShapeDtypeStruct((_BATCH, _DPAD), jnp.float32),
        mesh=plsc.VectorSubcoreMesh(core_axis_name="c", subcore_axis_name="s",
                                    num_cores=2, num_subcores=16),
        scratch_types=[
            pltpu.VMEM((_BPT, _SIZE), jnp.float32),
            pltpu.VMEM((_BPT, _SIZE), jnp.int32),
            pltpu.VMEM((_SIZE, _NWORDS), jnp.int32),
            pltpu.VMEM((_LEVELS, _NWORDS), jnp.int32),
            pltpu.VMEM((_BPT, _DPAD), jnp.float32),
        ],
    )(_sc_encode_body)


def _pack_bits(bits):
    r = bits.shape[0]
    padded = jnp.pad(bits, ((0, 0), (0, _DPAD - _DIMS)))
    shifted = padded.reshape(r, _NWORDS, 32).astype(jnp.uint32) << jnp.arange(
        32, dtype=jnp.uint32)
    words = jnp.sum(shifted, axis=-1, dtype=jnp.uint32)
    return lax.bitcast_convert_type(words, jnp.int32)


@jax.jit
def kernel(x, keys_weight, values_weight):
    kinv = _pack_bits(keys_weight <= 0)
    vbit = _pack_bits(values_weight > 0)
    out = _sc_encode()(x, kinv, vbit)
    return out[:, :_DIMS]
